# baseline (speedup 1.0000x reference)
import functools

import jax
import jax.numpy as jnp
from jax import lax
from jax.experimental import pallas as pl
from jax.experimental.pallas import tpu as pltpu
from jax.experimental.pallas import tpu_sc as plsc

EMBED = 64
WIDE = 128
SCALE = 8.0
NC = 2
NS = 16
NW = NC * NS
CHUNK = 128
LANES = 16


@functools.partial(jax.jit, static_argnames=("n_chunks",))
def _emb_lookup(tok3, table_wide, n_chunks):
    total = NW * n_chunks * CHUNK
    mesh = plsc.VectorSubcoreMesh(core_axis_name="c", subcore_axis_name="s")

    @functools.partial(
        pl.kernel,
        mesh=mesh,
        out_type=jax.ShapeDtypeStruct((total, EMBED), jnp.float32),
        scratch_types=[
            pltpu.VMEM((n_chunks, CHUNK), jnp.int32),
            pltpu.VMEM((2, CHUNK, WIDE), jnp.float32),
            pltpu.VMEM((2, CHUNK, EMBED), jnp.float32),
            pltpu.SemaphoreType.DMA,
            pltpu.SemaphoreType.DMA,
            pltpu.SemaphoreType.DMA,
            pltpu.SemaphoreType.DMA,
        ],
    )
    def body(tok_hbm, table_hbm, out_hbm, idx_v, wide_v, comp_v, g0, g1, w0, w1):
        gsem = (g0, g1)
        wsem = (w0, w1)
        wid = lax.axis_index("s") * NC + lax.axis_index("c")
        base = wid * (n_chunks * CHUNK)
        pltpu.sync_copy(tok_hbm.at[wid], idx_v)

        pltpu.async_copy(table_hbm.at[idx_v.at[0]], wide_v.at[0], gsem[0])

        @pl.loop(0, n_chunks, step=2)
        def outer(j0):
            for b in range(2):
                j = j0 + b
                other = 1 - b

                @pl.when(j + 1 < n_chunks)
                def _():
                    pltpu.async_copy(
                        table_hbm.at[idx_v.at[j + 1]], wide_v.at[other],
                        gsem[other],
                    )

                pltpu.make_async_copy(
                    table_hbm.at[pl.ds(0, CHUNK)], wide_v.at[b], gsem[b]
                ).wait()

                @pl.when(j >= 2)
                def _():
                    pltpu.make_async_copy(
                        comp_v.at[b], out_hbm.at[pl.ds(0, CHUNK)], wsem[b]
                    ).wait()

                @plsc.parallel_loop(0, CHUNK, 1, unroll=8)
                def scale_row(r):
                    for d in range(EMBED // LANES):
                        sl = pl.ds(d * LANES, LANES)
                        comp_v[b, r, sl] = wide_v[b, r, sl] * SCALE

                pltpu.async_copy(
                    comp_v.at[b],
                    out_hbm.at[pl.ds(base + j * CHUNK, CHUNK)],
                    wsem[b],
                )

        for b in range(2):
            pltpu.make_async_copy(
                comp_v.at[b], out_hbm.at[pl.ds(0, CHUNK)], wsem[b]
            ).wait()

    return body(tok3, table_wide)


def kernel(tokens, table):
    b, s = tokens.shape
    total = b * s
    n_chunks = total // (NW * CHUNK)
    tok3 = tokens.astype(jnp.int32).reshape(NW, n_chunks, CHUNK)
    table_wide = jnp.pad(table, ((0, 0), (0, WIDE - EMBED)))
    out = _emb_lookup(tok3, table_wide, n_chunks)
    return out.reshape(b, s, EMBED)


# speedup vs baseline: 1.0139x; 1.0139x over previous
import functools

import jax
import jax.numpy as jnp
from jax import lax
from jax.experimental import pallas as pl
from jax.experimental.pallas import tpu as pltpu
from jax.experimental.pallas import tpu_sc as plsc

EMBED = 64
WIDE = 128
SCALE = 8.0
NC = 2
NS = 16
NW = NC * NS
CHUNK = 128
LANES = 16


def _transpose_scale(src_v, dst_v, n_rows, scale, n_cols):
    lane = lax.iota(jnp.int32, LANES)

    @plsc.parallel_loop(0, n_rows, 1, unroll=4)
    def rows(r):
        rcol = jnp.broadcast_to(r, (LANES,)).astype(jnp.int32)
        for d in range(n_cols // LANES):
            vec = src_v[r, pl.ds(d * LANES, LANES)]
            plsc.store_scatter(dst_v, [lane + d * LANES, rcol], vec * scale)


@functools.partial(jax.jit, static_argnames=("seq", "batch"))
def _emb_lookup(tok_t, table_wide, seq, batch):
    mesh = plsc.VectorSubcoreMesh(core_axis_name="c", subcore_axis_name="s")

    @functools.partial(
        pl.kernel,
        mesh=mesh,
        out_type=jax.ShapeDtypeStruct((seq, EMBED, batch), jnp.float32),
        scratch_types=[
            pltpu.VMEM((seq, CHUNK), jnp.int32),
            pltpu.VMEM((2, CHUNK, WIDE), jnp.float32),
            pltpu.VMEM((2, EMBED, CHUNK + 1), jnp.float32),
            pltpu.SemaphoreType.DMA,
            pltpu.SemaphoreType.DMA,
            pltpu.SemaphoreType.DMA,
            pltpu.SemaphoreType.DMA,
        ],
        compiler_params=pltpu.CompilerParams(
            use_tc_tiling_on_sc=False, needs_layout_passes=False
        ),
    )
    def body(tok_hbm, table_hbm, out_hbm, idx_v, wide_v, trans_v, g0, g1, w0, w1):
        gsem = (g0, g1)
        wsem = (w0, w1)
        wid = lax.axis_index("s") * NC + lax.axis_index("c")
        col0 = wid * CHUNK
        pltpu.sync_copy(tok_hbm.at[:, pl.ds(col0, CHUNK)], idx_v)

        pltpu.async_copy(table_hbm.at[idx_v.at[0]], wide_v.at[0], gsem[0])

        @pl.loop(0, seq, step=2)
        def outer(j0):
            for b in range(2):
                j = j0 + b
                other = 1 - b

                @pl.when(j + 1 < seq)
                def _():
                    pltpu.async_copy(
                        table_hbm.at[idx_v.at[j + 1]], wide_v.at[other],
                        gsem[other],
                    )

                pltpu.make_async_copy(
                    table_hbm.at[pl.ds(0, CHUNK)], wide_v.at[b], gsem[b]
                ).wait()

                @pl.when(j >= 2)
                def _():
                    pltpu.make_async_copy(
                        trans_v.at[b, :, pl.ds(0, CHUNK)],
                        out_hbm.at[0, :, pl.ds(0, CHUNK)],
                        wsem[b],
                    ).wait()

                _transpose_scale(
                    wide_v.at[b], trans_v.at[b], CHUNK, SCALE, n_cols=EMBED
                )

                pltpu.async_copy(
                    trans_v.at[b, :, pl.ds(0, CHUNK)],
                    out_hbm.at[j, :, pl.ds(col0, CHUNK)],
                    wsem[b],
                )

        for b in range(2):
            pltpu.make_async_copy(
                trans_v.at[b, :, pl.ds(0, CHUNK)],
                out_hbm.at[0, :, pl.ds(0, CHUNK)],
                wsem[b],
            ).wait()

    return body(tok_t, table_wide)


def kernel(tokens, table):
    b, s = tokens.shape
    tok_t = tokens.T.astype(jnp.int32)
    table_wide = jnp.pad(table, ((0, 0), (0, WIDE - EMBED)))
    out = _emb_lookup(tok_t, table_wide, s, b)
    return out.transpose(2, 0, 1)
